# baseline (speedup 1.0000x reference)
import functools

import jax
import jax.numpy as jnp
from jax import lax
from jax.experimental import pallas as pl
from jax.experimental.pallas import tpu as pltpu
from jax.experimental.pallas import tpu_sc as plsc

NC = 2
NS = 16
C = 128


def _sc_aggregate(x_all, src_stack, dst_stack, z128, o128, n_dst, d, ch):
    epc = ch * C
    acc_chunks = -(-(n_dst + 1) // C)
    acc_rows = acc_chunks * C
    full = n_dst // C
    rem = n_dst % C
    assert rem % 8 == 0

    mesh = plsc.VectorSubcoreMesh(
        core_axis_name="c", subcore_axis_name="s",
        num_cores=NC, num_subcores=NS)

    @functools.partial(
        pl.kernel,
        out_type=(
            jax.ShapeDtypeStruct((2, n_dst, d), jnp.float32),
            jax.ShapeDtypeStruct((2, n_dst, d), jnp.float32),
        ),
        mesh=mesh,
        scratch_types=[
            [pltpu.VMEM((C,), jnp.int32)] * 4,
            [pltpu.VMEM((C,), jnp.int32)] * 4,
            pltpu.VMEM((C, d), jnp.float32),
            pltpu.VMEM((C, d), jnp.float32),
            pltpu.VMEM_SHARED((acc_rows, d), jnp.float32),
            pltpu.SemaphoreType.DMA,
            pltpu.SemaphoreType.DMA,
            pltpu.SemaphoreType.DMA,
            pltpu.SemaphoreType.DMA,
            [pltpu.SemaphoreType.DMA] * 4,
        ],
    )
    def body(x_all_h, src_h, dst_h, z128_h, o128_h,
             sum_out, cnt_out, isl, idl,
             rows0, rows1, accum, gs0, gs1, ss0, ss1, islot):
        cid = lax.axis_index("c")
        sid = lax.axis_index("s")
        rows = (rows0, rows1)
        gsem = (gs0, gs1)
        ssem = (ss0, ss1)

        def zero_accum():
            for j in range(-(-acc_chunks // NS)):
                k = sid + NS * j
                if (j + 1) * NS <= acc_chunks:
                    pltpu.sync_copy(rows0, accum.at[pl.ds(k * C, C)])
                else:
                    @pl.when(k < acc_chunks)
                    def _():
                        pltpu.sync_copy(rows0, accum.at[pl.ds(k * C, C)])

        def copy_out(dst_hbm, rel, zero_after=False):
            def chunk(r0, nr):
                pltpu.sync_copy(accum.at[pl.ds(r0, nr)], rows0.at[pl.ds(0, nr)])
                pltpu.sync_copy(rows0.at[pl.ds(0, nr)],
                                dst_hbm.at[rel, pl.ds(r0, nr)])
                if zero_after:
                    pltpu.sync_copy(rows1.at[pl.ds(0, nr)],
                                    accum.at[pl.ds(r0, nr)])

            for j in range(-(-full // NS)):
                m = sid + NS * j
                if (j + 1) * NS <= full:
                    chunk(m * C, C)
                else:
                    @pl.when(m < full)
                    def _():
                        chunk(m * C, C)
            if rem:
                @pl.when(sid == NS - 1)
                def _():
                    chunk(full * C, rem)

        tbase = sid * epc

        def load_slot(rel, k, i):
            pltpu.async_copy(src_h.at[rel, pl.ds(tbase + i * C, C)],
                             isl[k], islot[k])
            pltpu.async_copy(dst_h.at[rel, pl.ds(tbase + i * C, C)],
                             idl[k], islot[k])

        def wait_slot(rel, k, src_too=True):
            if src_too:
                pltpu.make_async_copy(src_h.at[rel, pl.ds(tbase, C)],
                                      isl[k], islot[k]).wait()
            pltpu.make_async_copy(dst_h.at[rel, pl.ds(tbase, C)],
                                  idl[k], islot[k]).wait()

        def load_slot_sync(rel, k, i):
            pltpu.sync_copy(src_h.at[rel, pl.ds(tbase + i * C, C)], isl[k])
            pltpu.sync_copy(dst_h.at[rel, pl.ds(tbase + i * C, C)], idl[k])

        def sums_pipeline(rel):
            for k in range(4):
                load_slot_sync(rel, k, k)
            pltpu.async_copy(x_all_h.at[isl[0]], rows0, gs0)

            def pbody(j, _):
                c0 = 4 * j
                for m in range(4):
                    c = c0 + m
                    kn = (m + 1) % 4
                    b, bn = m % 2, (m + 1) % 2

                    @pl.when(c + 1 >= 4)
                    def _():
                        wait_slot(rel, kn)

                    @pl.when(c >= 1)
                    def _():
                        pltpu.make_async_copy(
                            rows[bn], accum.at[idl[(m + 3) % 4]],
                            ssem[bn]).wait()

                    @pl.when(jnp.logical_and(c >= 1, c + 3 < ch))
                    def _():
                        load_slot(rel, (m + 3) % 4, c + 3)
                    pltpu.async_copy(x_all_h.at[isl[kn]], rows[bn], gsem[bn])
                    pltpu.make_async_copy(x_all_h.at[isl[m]],
                                          rows[b], gsem[b]).wait()
                    pltpu.async_copy(rows[b], accum.at[idl[m]], ssem[b],
                                     add=True)
                return 0

            lax.fori_loop(0, (ch - 1) // 4, pbody, 0)
            pltpu.make_async_copy(rows1, accum.at[idl[3]], ss1).wait()
            pltpu.make_async_copy(x_all_h.at[isl[0]], rows0, gs0).wait()
            pltpu.async_copy(rows0, accum.at[idl[0]], ss0, add=True)
            pltpu.make_async_copy(rows0, accum.at[idl[0]], ss0).wait()

        def counts_pipeline(rel):
            for k in range(4):
                load_slot_sync(rel, k, k)
            pltpu.async_copy(rows1, accum.at[idl[0]], ss0, add=True)

            def pbody2(j, _):
                for m in range(4):
                    c = 4 * j + 1 + m
                    k = (1 + m) % 4
                    p = (1 + m) % 2

                    @pl.when(c >= 4)
                    def _():
                        wait_slot(rel, k, src_too=False)
                    pltpu.async_copy(rows1, accum.at[idl[k]], ssem[p],
                                     add=True)
                    pltpu.make_async_copy(rows1, accum.at[idl[(k + 3) % 4]],
                                          ssem[1 - p]).wait()

                    @pl.when(c + 3 < ch)
                    def _():
                        pltpu.async_copy(
                            dst_h.at[rel, pl.ds(tbase + (c + 3) * C, C)],
                            idl[(k + 3) % 4], islot[(k + 3) % 4])
                return 0

            lax.fori_loop(0, (ch - 1) // 4, pbody2, 0)
            pltpu.make_async_copy(rows1, accum.at[idl[0]], ss0).wait()

        @pl.when(cid == 0)
        def _():
            pltpu.sync_copy(z128_h, rows0)
            zero_accum()
            plsc.subcore_barrier()
            for rel in range(2):
                sums_pipeline(rel)
                pltpu.sync_copy(z128_h, rows1)
                plsc.subcore_barrier()
                copy_out(sum_out, rel, zero_after=(rel == 0))
                plsc.subcore_barrier()

        @pl.when(cid == 1)
        def _():
            pltpu.sync_copy(z128_h, rows0)
            zero_accum()
            pltpu.sync_copy(o128_h, rows1)
            plsc.subcore_barrier()
            for rel in range(2):
                counts_pipeline(rel)
                pltpu.sync_copy(z128_h, rows1)
                plsc.subcore_barrier()
                copy_out(cnt_out, rel, zero_after=(rel == 0))
                plsc.subcore_barrier()
                if rel == 0:
                    pltpu.sync_copy(o128_h, rows1)

    return body(x_all, src_stack, dst_stack, z128, o128)


def _tc_dense(summed, cnt, x_all, wl, bl, wr, g, b):
    _, n, d = summed.shape
    bs = 1000
    nb = n // bs

    def body(s_ref, c_ref, x_ref, wl_ref, bl_ref, wr_ref, g_ref, b_ref, o_ref):
        mean = s_ref[0] / jnp.maximum(c_ref[0], 1.0)
        x = x_ref[...]
        h = (jnp.dot(mean, wl_ref[0], preferred_element_type=jnp.float32,
                     precision=lax.Precision.HIGHEST)
             + bl_ref[0]
             + jnp.dot(x, wr_ref[0], preferred_element_type=jnp.float32,
                       precision=lax.Precision.HIGHEST))
        t = h + x
        mu = jnp.mean(t, axis=1, keepdims=True)
        var = jnp.mean(jnp.square(t - mu), axis=1, keepdims=True)
        o_ref[0] = (t - mu) * lax.rsqrt(var + 1e-5) * g_ref[0] + b_ref[0]

    return pl.pallas_call(
        body,
        grid=(2, nb),
        in_specs=[
            pl.BlockSpec((1, bs, d), lambda i, j: (i, j, 0)),
            pl.BlockSpec((1, bs, d), lambda i, j: (i, j, 0)),
            pl.BlockSpec((bs, d), lambda i, j: ((1 - i) * nb + j, 0)),
            pl.BlockSpec((1, d, d), lambda i, j: (i, 0, 0)),
            pl.BlockSpec((1, 1, d), lambda i, j: (i, 0, 0)),
            pl.BlockSpec((1, d, d), lambda i, j: (i, 0, 0)),
            pl.BlockSpec((1, 1, d), lambda i, j: (i, 0, 0)),
            pl.BlockSpec((1, 1, d), lambda i, j: (i, 0, 0)),
        ],
        out_specs=pl.BlockSpec((1, bs, d), lambda i, j: (i, j, 0)),
        out_shape=jax.ShapeDtypeStruct((2, n, d), jnp.float32),
    )(summed, cnt, x_all, wl, bl, wr, g, b)


def kernel(x_sample, x_gene, edge_index_expresses, edge_index_expressed_by,
           W_l_sg, b_l_sg, W_r_sg, W_l_gs, b_l_gs, W_r_gs,
           ln_g_sample, ln_b_sample, ln_g_gene, ln_b_gene):
    n_s, d = x_sample.shape
    n_g = x_gene.shape[0]
    e = edge_index_expresses.shape[1]
    assert n_s == n_g and n_s % 8 == 0

    src0 = edge_index_expressed_by[0].astype(jnp.int32)
    dst0 = edge_index_expressed_by[1].astype(jnp.int32)
    src1 = edge_index_expresses[0].astype(jnp.int32) + n_g
    dst1 = edge_index_expresses[1].astype(jnp.int32)
    x_all = jnp.concatenate([x_gene, x_sample], axis=0)

    ch = -(-e // (NS * C))
    while ch % 4 != 1:
        ch += 1
    epad = NS * ch * C
    trash = n_s
    pad = epad - e
    src_stack = jnp.stack([
        jnp.pad(src0, (0, pad)),
        jnp.pad(src1, (0, pad), constant_values=n_g),
    ])
    dst_stack = jnp.stack([
        jnp.pad(dst0, (0, pad), constant_values=trash),
        jnp.pad(dst1, (0, pad), constant_values=trash),
    ])
    z128 = jnp.zeros((C, d), jnp.float32)
    o128 = jnp.ones((C, d), jnp.float32)

    summed, cnt = _sc_aggregate(x_all, src_stack, dst_stack, z128, o128,
                                n_s, d, ch)

    wl = jnp.stack([W_l_gs, W_l_sg])
    bl = jnp.stack([b_l_gs, b_l_sg])[:, None, :]
    wr = jnp.stack([W_r_gs, W_r_sg])
    g = jnp.stack([ln_g_sample, ln_g_gene])[:, None, :]
    b = jnp.stack([ln_b_sample, ln_b_gene])[:, None, :]
    return _tc_dense(summed, cnt, x_all, wl, bl, wr, g, b)
